# Initial kernel scaffold; baseline (speedup 1.0000x reference)
import jax
import jax.numpy as jnp
from jax.experimental import pallas as pl


def kernel(X, P_logit, Q):
    raise NotImplementedError("write your pallas kernel here")


# speedup vs baseline: 11.3479x; 11.3479x over previous
import functools

import jax
import jax.numpy as jnp
from jax.experimental import pallas as pl

K = 12
A = 21
U = 32
TD = 4
ND = 100
FD = 6
LD = 334
PD = LD - K + 1
EPS = 1e-06


def _r_kernel(p_ref, q_ref, r_ref):
    p_logit = p_ref[...]
    m = jnp.max(p_logit, axis=1, keepdims=True)
    e = jnp.exp(p_logit - m)
    p = e / jnp.sum(e, axis=1, keepdims=True)
    q = q_ref[...].reshape(1, A, 1)
    r_ref[...] = jnp.log(jnp.maximum(p / q, EPS))


def _conv_kernel(x_ref, r_ref, z_ref, s_ref):
    f = pl.program_id(1)
    x = x_ref[0, 0]
    rmat = r_ref[...].reshape(K * A, U)
    patches = jnp.concatenate(
        [x[k:k + PD, :] for k in range(K)], axis=1)
    z = jnp.dot(patches, rmat, preferred_element_type=jnp.float32)
    z_ref[0, 0] = z
    m = jnp.max(z, axis=0)

    @pl.when(f == 0)
    def _init():
        s_ref[0, 0] = m

    @pl.when(f != 0)
    def _acc():
        s_ref[0, 0] = jnp.maximum(s_ref[0, 0], m)


@functools.partial(jax.jit, static_argnames=("interpret",))
def kernel(X, P_logit, Q, interpret=False):
    R = pl.pallas_call(
        _r_kernel,
        out_shape=jax.ShapeDtypeStruct((K, A, U), jnp.float32),
        interpret=interpret,
    )(P_logit, Q)

    xb = X.reshape(TD * ND, FD, LD, A)
    grid = (TD * ND, FD)
    Z, S = pl.pallas_call(
        _conv_kernel,
        grid=grid,
        in_specs=[
            pl.BlockSpec((1, 1, LD, A), lambda i, j: (i, j, 0, 0)),
            pl.BlockSpec((K, A, U), lambda i, j: (0, 0, 0)),
        ],
        out_specs=[
            pl.BlockSpec((1, 1, PD, U), lambda i, j: (i, j, 0, 0)),
            pl.BlockSpec((1, 1, U), lambda i, j: (i, 0, 0)),
        ],
        out_shape=[
            jax.ShapeDtypeStruct((TD * ND, FD, PD, U), jnp.float32),
            jax.ShapeDtypeStruct((TD * ND, 1, U), jnp.float32),
        ],
        interpret=interpret,
    )(xb, R)
    S = S.reshape(TD, ND, U)
    Z = Z.reshape(TD, ND, FD, PD, U)
    return S, R, Z


# speedup vs baseline: 11.8736x; 1.0463x over previous
import functools

import jax
import jax.numpy as jnp
from jax.experimental import pallas as pl

K = 12
A = 21
U = 32
TD = 4
ND = 100
FD = 6
LD = 334
PD = LD - K + 1
EPS = 1e-06


def _r_kernel(p_ref, q_ref, r_ref):
    p_logit = p_ref[...]
    m = jnp.max(p_logit, axis=1, keepdims=True)
    e = jnp.exp(p_logit - m)
    p = e / jnp.sum(e, axis=1, keepdims=True)
    q = q_ref[...].reshape(1, A, 1)
    r_ref[...] = jnp.log(jnp.maximum(p / q, EPS))


def _conv_kernel(x_ref, r_ref, z_ref, s_ref):
    f = pl.program_id(2)
    x = x_ref[0, 0, 0]
    rmat = r_ref[...].reshape(K * A, U)
    patches = jnp.concatenate(
        [x[k:k + PD, :] for k in range(K)], axis=1)
    z = jnp.dot(patches, rmat, preferred_element_type=jnp.float32)
    z_ref[0, 0, 0] = z
    m = jnp.max(z, axis=0)

    @pl.when(f == 0)
    def _init():
        s_ref[0, 0, 0] = m

    @pl.when(f != 0)
    def _acc():
        s_ref[0, 0, 0] = jnp.maximum(s_ref[0, 0, 0], m)


@functools.partial(jax.jit, static_argnames=("interpret",))
def kernel(X, P_logit, Q, interpret=False):
    R = pl.pallas_call(
        _r_kernel,
        out_shape=jax.ShapeDtypeStruct((K, A, U), jnp.float32),
        interpret=interpret,
    )(P_logit, Q)

    grid = (TD, ND, FD)
    Z, S = pl.pallas_call(
        _conv_kernel,
        grid=grid,
        in_specs=[
            pl.BlockSpec((1, 1, 1, LD, A), lambda t, n, f: (t, n, f, 0, 0)),
            pl.BlockSpec((K, A, U), lambda t, n, f: (0, 0, 0)),
        ],
        out_specs=[
            pl.BlockSpec((1, 1, 1, PD, U), lambda t, n, f: (t, n, f, 0, 0)),
            pl.BlockSpec((1, 1, 1, U), lambda t, n, f: (t, n, 0, 0)),
        ],
        out_shape=[
            jax.ShapeDtypeStruct((TD, ND, FD, PD, U), jnp.float32),
            jax.ShapeDtypeStruct((TD, ND, 1, U), jnp.float32),
        ],
        interpret=interpret,
    )(X, R)
    return S.reshape(TD, ND, U), R, Z


# speedup vs baseline: 18.3035x; 1.5415x over previous
import functools

import jax
import jax.numpy as jnp
from jax.experimental import pallas as pl

K = 12
A = 21
U = 32
TD = 4
ND = 100
FD = 6
LD = 334
PD = LD - K + 1
EPS = 1e-06


def _r_kernel(p_ref, q_ref, r_ref):
    p_logit = p_ref[...]
    m = jnp.max(p_logit, axis=1, keepdims=True)
    e = jnp.exp(p_logit - m)
    p = e / jnp.sum(e, axis=1, keepdims=True)
    q = q_ref[...].reshape(1, A, 1)
    r_ref[...] = jnp.log(jnp.maximum(p / q, EPS))


def _conv_kernel(x_ref, r_ref, z_ref, s_ref):
    rmat = r_ref[...].reshape(K * A, U)
    m = None
    for f in range(FD):
        x = x_ref[0, 0, f]
        patches = jnp.concatenate(
            [x[k:k + PD, :] for k in range(K)], axis=1)
        z = jnp.dot(patches, rmat, preferred_element_type=jnp.float32)
        z_ref[0, 0, f] = z
        zm = jnp.max(z, axis=0)
        m = zm if m is None else jnp.maximum(m, zm)
    s_ref[0, 0, 0] = m


@functools.partial(jax.jit, static_argnames=("interpret",))
def kernel(X, P_logit, Q, interpret=False):
    R = pl.pallas_call(
        _r_kernel,
        out_shape=jax.ShapeDtypeStruct((K, A, U), jnp.float32),
        interpret=interpret,
    )(P_logit, Q)

    grid = (TD, ND)
    Z, S = pl.pallas_call(
        _conv_kernel,
        grid=grid,
        in_specs=[
            pl.BlockSpec((1, 1, FD, LD, A), lambda t, n: (t, n, 0, 0, 0)),
            pl.BlockSpec((K, A, U), lambda t, n: (0, 0, 0)),
        ],
        out_specs=[
            pl.BlockSpec((1, 1, FD, PD, U), lambda t, n: (t, n, 0, 0, 0)),
            pl.BlockSpec((1, 1, 1, U), lambda t, n: (t, n, 0, 0)),
        ],
        out_shape=[
            jax.ShapeDtypeStruct((TD, ND, FD, PD, U), jnp.float32),
            jax.ShapeDtypeStruct((TD, ND, 1, U), jnp.float32),
        ],
        interpret=interpret,
    )(X, R)
    return S.reshape(TD, ND, U), R, Z


# speedup vs baseline: 18.4770x; 1.0095x over previous
import functools

import jax
import jax.numpy as jnp
from jax.experimental import pallas as pl

K = 12
A = 21
U = 32
TD = 4
ND = 100
FD = 6
LD = 334
PD = LD - K + 1
EPS = 1e-06


def _r_kernel(p_ref, q_ref, r_ref):
    p_logit = p_ref[...]
    m = jnp.max(p_logit, axis=1, keepdims=True)
    e = jnp.exp(p_logit - m)
    p = e / jnp.sum(e, axis=1, keepdims=True)
    q = q_ref[...].reshape(1, A, 1)
    r_ref[...] = jnp.log(jnp.maximum(p / q, EPS))


NB = 2


def _conv_kernel(x_ref, r_ref, z_ref, s_ref):
    rmat = r_ref[...].reshape(K * A, U)
    for nb in range(NB):
        m = None
        for f in range(FD):
            x = x_ref[0, nb, f]
            patches = jnp.concatenate(
                [x[k:k + PD, :] for k in range(K)], axis=1)
            z = jnp.dot(patches, rmat, preferred_element_type=jnp.float32)
            z_ref[0, nb, f] = z
            zm = jnp.max(z, axis=0)
            m = zm if m is None else jnp.maximum(m, zm)
        s_ref[0, nb, 0] = m


@functools.partial(jax.jit, static_argnames=("interpret",))
def kernel(X, P_logit, Q, interpret=False):
    R = pl.pallas_call(
        _r_kernel,
        out_shape=jax.ShapeDtypeStruct((K, A, U), jnp.float32),
        interpret=interpret,
    )(P_logit, Q)

    grid = (TD, ND // NB)
    Z, S = pl.pallas_call(
        _conv_kernel,
        grid=grid,
        in_specs=[
            pl.BlockSpec((1, NB, FD, LD, A), lambda t, n: (t, n, 0, 0, 0)),
            pl.BlockSpec((K, A, U), lambda t, n: (0, 0, 0)),
        ],
        out_specs=[
            pl.BlockSpec((1, NB, FD, PD, U), lambda t, n: (t, n, 0, 0, 0)),
            pl.BlockSpec((1, NB, 1, U), lambda t, n: (t, n, 0, 0)),
        ],
        out_shape=[
            jax.ShapeDtypeStruct((TD, ND, FD, PD, U), jnp.float32),
            jax.ShapeDtypeStruct((TD, ND, 1, U), jnp.float32),
        ],
        interpret=interpret,
    )(X, R)
    return S.reshape(TD, ND, U), R, Z


# speedup vs baseline: 23.5387x; 1.2739x over previous
import functools

import jax
import jax.numpy as jnp
from jax.experimental import pallas as pl
from jax.experimental.pallas import tpu as pltpu

K = 12
A = 21
U = 32
TD = 4
ND = 100
FD = 6
LD = 334
PD = LD - K + 1
EPS = 1e-06

AP = 32
KA = K * AP
NB = 2


def _r_kernel(p_ref, q_ref, r_ref, w_ref):
    p_logit = p_ref[...]
    m = jnp.max(p_logit, axis=1, keepdims=True)
    e = jnp.exp(p_logit - m)
    p = e / jnp.sum(e, axis=1, keepdims=True)
    q = q_ref[...].reshape(1, A, 1)
    r = jnp.log(jnp.maximum(p / q, EPS))
    r_ref[...] = r
    rp = jnp.concatenate(
        [r, jnp.zeros((K, AP - A, U), jnp.float32)], axis=1)
    w_ref[...] = rp.transpose(2, 0, 1).reshape(U, KA)


def _conv_kernel(xt_ref, w_ref, zt_ref, s_ref, patches_ref):
    @pl.when((pl.program_id(0) == 0) & (pl.program_id(1) == 0))
    def _zero():
        patches_ref[...] = jnp.zeros((2, KA, PD), jnp.float32)

    w = w_ref[...]
    for nb in range(NB):
        m = None
        for f in range(FD):
            buf = (nb * FD + f) % 2
            xt = xt_ref[0, nb, f]
            for k in range(K):
                patches_ref[buf, AP * k:AP * k + A, :] = xt[:, k:k + PD]
            zt = jax.lax.dot_general(
                w, patches_ref[buf], (((1,), (0,)), ((), ())),
                preferred_element_type=jnp.float32)
            zt_ref[0, nb, f] = zt
            zm = jnp.max(zt, axis=1)
            m = zm if m is None else jnp.maximum(m, zm)
        s_ref[0, nb, 0] = m


@functools.partial(jax.jit, static_argnames=("interpret",))
def kernel(X, P_logit, Q, interpret=False):
    R, W = pl.pallas_call(
        _r_kernel,
        out_shape=[
            jax.ShapeDtypeStruct((K, A, U), jnp.float32),
            jax.ShapeDtypeStruct((U, KA), jnp.float32),
        ],
        interpret=interpret,
    )(P_logit, Q)

    XT = jnp.swapaxes(X, 3, 4)
    grid = (TD, ND // NB)
    ZT, S = pl.pallas_call(
        _conv_kernel,
        grid=grid,
        in_specs=[
            pl.BlockSpec((1, NB, FD, A, LD), lambda t, n: (t, n, 0, 0, 0)),
            pl.BlockSpec((U, KA), lambda t, n: (0, 0)),
        ],
        out_specs=[
            pl.BlockSpec((1, NB, FD, U, PD), lambda t, n: (t, n, 0, 0, 0)),
            pl.BlockSpec((1, NB, 1, U), lambda t, n: (t, n, 0, 0)),
        ],
        out_shape=[
            jax.ShapeDtypeStruct((TD, ND, FD, U, PD), jnp.float32),
            jax.ShapeDtypeStruct((TD, ND, 1, U), jnp.float32),
        ],
        scratch_shapes=[pltpu.VMEM((2, KA, PD), jnp.float32)],
        interpret=interpret,
    )(XT, W)
    Z = jnp.swapaxes(ZT, 3, 4)
    return S.reshape(TD, ND, U), R, Z


# speedup vs baseline: 27.1439x; 1.1532x over previous
import functools

import jax
import jax.numpy as jnp
from jax.experimental import pallas as pl
from jax.experimental.pallas import tpu as pltpu

K = 12
A = 21
U = 32
TD = 4
ND = 100
FD = 6
LD = 334
PD = LD - K + 1
EPS = 1e-06

AP = 32
KA = K * AP
NB = 2


def _r_kernel(p_ref, q_ref, r_ref, w_ref):
    p_logit = p_ref[...]
    m = jnp.max(p_logit, axis=1, keepdims=True)
    e = jnp.exp(p_logit - m)
    p = e / jnp.sum(e, axis=1, keepdims=True)
    q = q_ref[...].reshape(1, A, 1)
    r = jnp.log(jnp.maximum(p / q, EPS))
    r_ref[...] = r
    rp = jnp.concatenate(
        [r, jnp.zeros((K, AP - A, U), jnp.float32)], axis=1)
    w_ref[...] = rp.transpose(2, 0, 1).reshape(U, KA)


def _conv_kernel(xt_ref, w_ref, zt_ref, s_ref, patches_ref):
    @pl.when((pl.program_id(0) == 0) & (pl.program_id(1) == 0))
    def _zero():
        patches_ref[...] = jnp.zeros((2, KA, PD), jnp.float32)

    w = w_ref[...]
    for nb in range(NB):
        m = None
        for f in range(FD):
            buf = (nb * FD + f) % 2
            xt = xt_ref[0, nb, f].T
            for k in range(K):
                patches_ref[buf, AP * k:AP * k + A, :] = xt[:, k:k + PD]
            zt = jax.lax.dot_general(
                w, patches_ref[buf], (((1,), (0,)), ((), ())),
                preferred_element_type=jnp.float32)
            zt_ref[0, nb, f] = zt
            zm = jnp.max(zt, axis=1)
            m = zm if m is None else jnp.maximum(m, zm)
        s_ref[0, nb, 0] = m


@functools.partial(jax.jit, static_argnames=("interpret",))
def kernel(X, P_logit, Q, interpret=False):
    R, W = pl.pallas_call(
        _r_kernel,
        out_shape=[
            jax.ShapeDtypeStruct((K, A, U), jnp.float32),
            jax.ShapeDtypeStruct((U, KA), jnp.float32),
        ],
        interpret=interpret,
    )(P_logit, Q)

    grid = (TD, ND // NB)
    ZT, S = pl.pallas_call(
        _conv_kernel,
        grid=grid,
        in_specs=[
            pl.BlockSpec((1, NB, FD, LD, A), lambda t, n: (t, n, 0, 0, 0)),
            pl.BlockSpec((U, KA), lambda t, n: (0, 0)),
        ],
        out_specs=[
            pl.BlockSpec((1, NB, FD, U, PD), lambda t, n: (t, n, 0, 0, 0)),
            pl.BlockSpec((1, NB, 1, U), lambda t, n: (t, n, 0, 0)),
        ],
        out_shape=[
            jax.ShapeDtypeStruct((TD, ND, FD, U, PD), jnp.float32),
            jax.ShapeDtypeStruct((TD, ND, 1, U), jnp.float32),
        ],
        scratch_shapes=[pltpu.VMEM((2, KA, PD), jnp.float32)],
        interpret=interpret,
    )(X, W)
    Z = jnp.swapaxes(ZT, 3, 4)
    return S.reshape(TD, ND, U), R, Z


# speedup vs baseline: 33.9176x; 1.2495x over previous
import functools

import jax
import jax.numpy as jnp
from jax.experimental import pallas as pl

K = 12
A = 21
U = 32
TD = 4
ND = 100
FD = 6
LD = 334
PD = LD - K + 1
EPS = 1e-06

KU = K * U
NB = 2


def _r_kernel(p_ref, q_ref, r_ref, w2_ref):
    p_logit = p_ref[...]
    m = jnp.max(p_logit, axis=1, keepdims=True)
    e = jnp.exp(p_logit - m)
    p = e / jnp.sum(e, axis=1, keepdims=True)
    q = q_ref[...].reshape(1, A, 1)
    r = jnp.log(jnp.maximum(p / q, EPS))
    r_ref[...] = r
    w2_ref[...] = r.transpose(0, 2, 1).reshape(KU, A)


def _conv_kernel(x_ref, w2_ref, zt_ref, s_ref):
    w2 = w2_ref[...]
    for nb in range(NB):
        m = None
        for f in range(FD):
            x = x_ref[0, nb, f]
            yt = jax.lax.dot_general(
                w2, x, (((1,), (1,)), ((), ())),
                preferred_element_type=jnp.float32)
            zt = yt[0:U, 0:PD]
            for k in range(1, K):
                zt = zt + yt[U * k:U * (k + 1), k:k + PD]
            zt_ref[0, nb, f] = zt
            zm = jnp.max(zt, axis=1)
            m = zm if m is None else jnp.maximum(m, zm)
        s_ref[0, nb, 0] = m


@functools.partial(jax.jit, static_argnames=("interpret",))
def kernel(X, P_logit, Q, interpret=False):
    R, W2 = pl.pallas_call(
        _r_kernel,
        out_shape=[
            jax.ShapeDtypeStruct((K, A, U), jnp.float32),
            jax.ShapeDtypeStruct((KU, A), jnp.float32),
        ],
        interpret=interpret,
    )(P_logit, Q)

    grid = (TD, ND // NB)
    ZT, S = pl.pallas_call(
        _conv_kernel,
        grid=grid,
        in_specs=[
            pl.BlockSpec((1, NB, FD, LD, A), lambda t, n: (t, n, 0, 0, 0)),
            pl.BlockSpec((KU, A), lambda t, n: (0, 0)),
        ],
        out_specs=[
            pl.BlockSpec((1, NB, FD, U, PD), lambda t, n: (t, n, 0, 0, 0)),
            pl.BlockSpec((1, NB, 1, U), lambda t, n: (t, n, 0, 0)),
        ],
        out_shape=[
            jax.ShapeDtypeStruct((TD, ND, FD, U, PD), jnp.float32),
            jax.ShapeDtypeStruct((TD, ND, 1, U), jnp.float32),
        ],
        interpret=interpret,
    )(X, W2)
    Z = jnp.swapaxes(ZT, 3, 4)
    return S.reshape(TD, ND, U), R, Z
